# baseline (device time: 20367 ns/iter reference)
import jax
import jax.numpy as jnp
from jax import lax
from jax.experimental import pallas as pl
from jax.experimental.pallas import tpu as pltpu

NCH = 16
NBLK = 4


def kernel(x, dy):
    k, d = x.shape
    _, f = dy.shape
    half = d // 2
    cw = f // NCH
    bw = f // NBLK
    cpb = NCH // NBLK

    def body(x_ref, dy_ref, out_ref,
             pmine_buf, psend_buf, yrecv_buf,
             ysend_sems, yrecv_sems):
        my_x = lax.axis_index("x")
        my_y = lax.axis_index("y")
        my_z = lax.axis_index("z")
        ypartner = (my_x, 1 - my_y, my_z)

        barrier = pltpu.get_barrier_semaphore()
        pl.semaphore_signal(
            barrier, inc=1, device_id=ypartner,
            device_id_type=pl.DeviceIdType.MESH,
        )
        pl.semaphore_wait(barrier, 1)

        def y_rdma(j):
            return pltpu.make_async_remote_copy(
                src_ref=psend_buf.at[j],
                dst_ref=yrecv_buf.at[j],
                send_sem=ysend_sems.at[j],
                recv_sem=yrecv_sems.at[j],
                device_id=ypartner,
                device_id_type=pl.DeviceIdType.MESH,
            )

        def impl(mine, theirs):
            for b in range(NBLK):
                p = lax.dot_general(
                    x_ref[...], dy_ref[:, b * bw:(b + 1) * bw],
                    dimension_numbers=(((0,), (0,)), ((), ())),
                    preferred_element_type=jnp.float32,
                )
                for i in range(cpb):
                    j = b * cpb + i
                    sl = p[:, i * cw:(i + 1) * cw]
                    pmine_buf[j] = sl[mine:mine + half]
                    psend_buf[j] = sl[theirs:theirs + half].astype(jnp.bfloat16)
                    y_rdma(j).start()

            for j in range(NCH):
                y_rdma(j).wait_recv()
                out_ref[:, j * cw:(j + 1) * cw] = (
                    pmine_buf[j] + yrecv_buf[j].astype(jnp.float32)
                )

            for j in range(NCH):
                y_rdma(j).wait_send()

        pl.when(my_y == 0)(lambda: impl(0, half))
        pl.when(my_y == 1)(lambda: impl(half, 0))

    return pl.pallas_call(
        body,
        out_shape=jax.ShapeDtypeStruct((half, f), jnp.float32),
        in_specs=[
            pl.BlockSpec(memory_space=pltpu.VMEM),
            pl.BlockSpec(memory_space=pltpu.VMEM),
        ],
        out_specs=pl.BlockSpec(memory_space=pltpu.VMEM),
        scratch_shapes=[
            pltpu.VMEM((NCH, half, cw), jnp.float32),
            pltpu.VMEM((NCH, half, cw), jnp.bfloat16),
            pltpu.VMEM((NCH, half, cw), jnp.bfloat16),
            pltpu.SemaphoreType.DMA((NCH,)),
            pltpu.SemaphoreType.DMA((NCH,)),
        ],
        compiler_params=pltpu.CompilerParams(collective_id=0),
    )(x, dy)
